# baseline (speedup 1.0000x reference)
import functools

import jax
import jax.numpy as jnp
from jax import lax
from jax.experimental import pallas as pl
from jax.experimental.pallas import tpu as pltpu
from jax.experimental.pallas import tpu_sc as plsc

_D = 256
_SEG_PAD = 8
_POS_PAD = 32


def _fused_kernel(emb_ref, seg_ref, pos_ref, st_ref, pt_ref, g_ref, b_ref,
                  out_ref):
    emb = emb_ref[...]
    seg = seg_ref[0]
    pos = pos_ref[0]

    r = emb.shape[0]
    oh_s = (seg == lax.broadcasted_iota(jnp.int32, (r, _SEG_PAD), 1))
    oh_p = (pos == lax.broadcasted_iota(jnp.int32, (r, _POS_PAD), 1))
    oh_s = oh_s.astype(jnp.bfloat16)
    oh_p = oh_p.astype(jnp.bfloat16)

    st = st_ref[...]
    pt = pt_ref[...]
    st_hi = st.astype(jnp.bfloat16)
    st_lo = (st - st_hi.astype(jnp.float32)).astype(jnp.bfloat16)
    pt_hi = pt.astype(jnp.bfloat16)
    pt_lo = (pt - pt_hi.astype(jnp.float32)).astype(jnp.bfloat16)

    x = emb
    x = x + jnp.dot(oh_s, st_hi, preferred_element_type=jnp.float32)
    x = x + jnp.dot(oh_s, st_lo, preferred_element_type=jnp.float32)
    x = x + jnp.dot(oh_p, pt_hi, preferred_element_type=jnp.float32)
    x = x + jnp.dot(oh_p, pt_lo, preferred_element_type=jnp.float32)

    mean = jnp.sum(x, axis=1, keepdims=True) * (1.0 / _D)
    meansq = jnp.sum(x * x, axis=1, keepdims=True) * (1.0 / _D)
    var = meansq - mean * mean
    inv = lax.rsqrt(var + 1e-5)
    gamma = g_ref[...]
    beta = b_ref[...]
    scale = inv * gamma
    shift = beta - (mean * inv) * gamma
    out_ref[...] = x * scale + shift


@functools.partial(jax.jit, static_argnames=("block_rows",))
def _run(emb2d, seg3d, pos3d, st_pad, pt_pad, gamma2d, beta2d,
         block_rows=512):
    n = emb2d.shape[0]
    nb = n // block_rows
    grid = (nb,)
    out = pl.pallas_call(
        _fused_kernel,
        grid=grid,
        in_specs=[
            pl.BlockSpec((block_rows, _D), lambda i: (i, 0)),
            pl.BlockSpec((1, block_rows, 1), lambda i: (i, 0, 0)),
            pl.BlockSpec((1, block_rows, 1), lambda i: (i, 0, 0)),
            pl.BlockSpec((_SEG_PAD, _D), lambda i: (0, 0)),
            pl.BlockSpec((_POS_PAD, _D), lambda i: (0, 0)),
            pl.BlockSpec((1, _D), lambda i: (0, 0)),
            pl.BlockSpec((1, _D), lambda i: (0, 0)),
        ],
        out_specs=pl.BlockSpec((block_rows, _D), lambda i: (i, 0)),
        out_shape=jax.ShapeDtypeStruct((n, _D), jnp.float32),
        compiler_params=pltpu.CompilerParams(
            dimension_semantics=("arbitrary",),
        ),
    )(emb2d, seg3d, pos3d, st_pad, pt_pad, gamma2d, beta2d)
    return out


_NC = 2
_NS = 16
_NW = _NC * _NS
_L = 16


def _sc_row_ln(seg_v, pos_v, st_v, pt_v, g_v, b_v, emb_v, out_v, xrow_v, r):
    ridx = jnp.full((_L,), r, jnp.int32)
    sv = plsc.load_gather(seg_v, [ridx])
    pv = plsc.load_gather(pos_v, [ridx])
    col0 = lax.broadcasted_iota(jnp.int32, (_L,), 0)
    acc = None
    accq = None
    for j in range(_D // _L):
        ds = pl.ds(j * _L, _L)
        cols = col0 + (j * _L)
        x = (emb_v[r, ds] + plsc.load_gather(st_v, [sv, cols])
             + plsc.load_gather(pt_v, [pv, cols]))
        xrow_v[ds] = x
        acc = x if acc is None else acc + x
        accq = x * x if accq is None else accq + x * x
    ssum = jnp.broadcast_to(jnp.sum(acc), (_L,))
    qsum = jnp.broadcast_to(jnp.sum(accq), (_L,))
    mean = ssum * (1.0 / _D)
    var = qsum * (1.0 / _D) - mean * mean + 1e-5
    i = plsc.bitcast(var, jnp.int32)
    y = plsc.bitcast(
        jnp.full((_L,), 0x5F3759DF, jnp.int32) - lax.shift_right_logical(i, 1),
        jnp.float32)
    for _ in range(4):
        y = y * (1.5 - 0.5 * var * y * y)
    for j in range(_D // _L):
        ds = pl.ds(j * _L, _L)
        x = xrow_v[ds]
        out_v[r, ds] = (x - mean) * y * g_v[ds] + b_v[ds]


def _make_sc_kernel(n_rows, chunk_rows):
    rows_per_w = n_rows // _NW
    n_chunks = rows_per_w // chunk_rows
    mesh = plsc.VectorSubcoreMesh(core_axis_name="c", subcore_axis_name="s")

    @functools.partial(
        pl.kernel,
        out_type=jax.ShapeDtypeStruct((n_rows, _D), jnp.float32),
        mesh=mesh,
        compiler_params=pltpu.CompilerParams(needs_layout_passes=False),
        scratch_types=[
            pltpu.VMEM((6, _D), jnp.float32),
            pltpu.VMEM((20, _D), jnp.float32),
            pltpu.VMEM((_D,), jnp.float32),
            pltpu.VMEM((_D,), jnp.float32),
            pltpu.VMEM((chunk_rows, _D), jnp.float32),
            pltpu.VMEM((chunk_rows,), jnp.int32),
            pltpu.VMEM((chunk_rows,), jnp.int32),
            pltpu.VMEM((chunk_rows, _D), jnp.float32),
            pltpu.VMEM((_D,), jnp.float32),
        ],
    )
    def sc_kern(emb_hbm, seg_hbm, pos_hbm, st_hbm, pt_hbm, g_hbm, b_hbm,
                out_hbm, st_v, pt_v, g_v, b_v, emb_v, seg_v, pos_v, out_v,
                xrow_v):
        wid = lax.axis_index("s") * _NC + lax.axis_index("c")
        base = wid * rows_per_w
        pltpu.sync_copy(st_hbm, st_v)
        pltpu.sync_copy(pt_hbm, pt_v)
        pltpu.sync_copy(g_hbm, g_v)
        pltpu.sync_copy(b_hbm, b_v)

        def chunk_body(ci, carry):
            r0 = base + ci * chunk_rows
            pltpu.sync_copy(emb_hbm.at[pl.ds(r0, chunk_rows)], emb_v)
            pltpu.sync_copy(seg_hbm.at[pl.ds(r0, chunk_rows)], seg_v)
            pltpu.sync_copy(pos_hbm.at[pl.ds(r0, chunk_rows)], pos_v)

            def row_body(r, inner):
                _sc_row_ln(seg_v, pos_v, st_v, pt_v, g_v, b_v, emb_v, out_v,
                           xrow_v, r)
                return inner

            lax.fori_loop(0, chunk_rows, row_body, 0)
            pltpu.sync_copy(out_v, out_hbm.at[pl.ds(r0, chunk_rows)])
            return carry

        lax.fori_loop(0, n_chunks, chunk_body, 0)

    return sc_kern


@jax.jit
def _run_sc(emb2d, seg1d, pos1d, seg_table, pos_table, gamma1d, beta1d):
    n = emb2d.shape[0]
    sc_kern = _make_sc_kernel(n, 64)
    return sc_kern(emb2d, seg1d, pos1d, seg_table, pos_table, gamma1d, beta1d)


def kernel(embeddings, segment_types, positions, seg_table, pos_table,
           ln_gamma, ln_beta):
    b, t, d = embeddings.shape
    n = b * t
    emb2d = embeddings.reshape(n, d)
    seg1d = segment_types.astype(jnp.int32).reshape(n)
    pos1d = positions.astype(jnp.int32).reshape(n)
    out = _run_sc(emb2d, seg1d, pos1d, seg_table, pos_table,
                  ln_gamma, ln_beta)
    return out.reshape(b, t, d)


def _kernel_tc(embeddings, segment_types, positions, seg_table, pos_table,
               ln_gamma, ln_beta):
    b, t, d = embeddings.shape
    n = b * t
    block_rows = 4096
    nb = n // block_rows

    emb2d = embeddings.reshape(n, d)
    seg3d = segment_types.astype(jnp.int32).reshape(nb, block_rows, 1)
    pos3d = positions.astype(jnp.int32).reshape(nb, block_rows, 1)
    st_pad = jnp.concatenate(
        [seg_table, jnp.zeros((_SEG_PAD - seg_table.shape[0], d), jnp.float32)])
    pt_pad = jnp.concatenate(
        [pos_table, jnp.zeros((_POS_PAD - pos_table.shape[0], d), jnp.float32)])
    gamma2d = ln_gamma.reshape(1, d)
    beta2d = ln_beta.reshape(1, d)

    out = _run(emb2d, seg3d, pos3d, st_pad, pt_pad, gamma2d, beta2d,
               block_rows=block_rows)
    return out.reshape(b, t, d)


# speedup vs baseline: 1.0354x; 1.0354x over previous
import functools

import jax
import jax.numpy as jnp
from jax import lax
from jax.experimental import pallas as pl
from jax.experimental.pallas import tpu as pltpu
from jax.experimental.pallas import tpu_sc as plsc

_D = 256
_SEG_PAD = 8
_POS_PAD = 32


def _fused_kernel(emb_ref, seg_ref, pos_ref, st_ref, pt_ref, g_ref, b_ref,
                  out_ref):
    emb = emb_ref[...]
    seg = seg_ref[0]
    pos = pos_ref[0]

    r = emb.shape[0]
    oh_s = (seg == lax.broadcasted_iota(jnp.int32, (r, _SEG_PAD), 1))
    oh_p = (pos == lax.broadcasted_iota(jnp.int32, (r, _POS_PAD), 1))
    oh_s = oh_s.astype(jnp.bfloat16)
    oh_p = oh_p.astype(jnp.bfloat16)

    st = st_ref[...]
    pt = pt_ref[...]
    st_hi = st.astype(jnp.bfloat16)
    st_lo = (st - st_hi.astype(jnp.float32)).astype(jnp.bfloat16)
    pt_hi = pt.astype(jnp.bfloat16)
    pt_lo = (pt - pt_hi.astype(jnp.float32)).astype(jnp.bfloat16)

    x = emb
    x = x + jnp.dot(oh_s, st_hi, preferred_element_type=jnp.float32)
    x = x + jnp.dot(oh_s, st_lo, preferred_element_type=jnp.float32)
    x = x + jnp.dot(oh_p, pt_hi, preferred_element_type=jnp.float32)
    x = x + jnp.dot(oh_p, pt_lo, preferred_element_type=jnp.float32)

    mean = jnp.sum(x, axis=1, keepdims=True) * (1.0 / _D)
    meansq = jnp.sum(x * x, axis=1, keepdims=True) * (1.0 / _D)
    var = meansq - mean * mean
    inv = lax.rsqrt(var + 1e-5)
    gamma = g_ref[...]
    beta = b_ref[...]
    scale = inv * gamma
    shift = beta - (mean * inv) * gamma
    out_ref[...] = x * scale + shift


@functools.partial(jax.jit, static_argnames=("block_rows",))
def _run(emb2d, seg3d, pos3d, st_pad, pt_pad, gamma2d, beta2d,
         block_rows=512):
    n = emb2d.shape[0]
    nb = n // block_rows
    grid = (nb,)
    out = pl.pallas_call(
        _fused_kernel,
        grid=grid,
        in_specs=[
            pl.BlockSpec((block_rows, _D), lambda i: (i, 0)),
            pl.BlockSpec((1, block_rows, 1), lambda i: (i, 0, 0)),
            pl.BlockSpec((1, block_rows, 1), lambda i: (i, 0, 0)),
            pl.BlockSpec((_SEG_PAD, _D), lambda i: (0, 0)),
            pl.BlockSpec((_POS_PAD, _D), lambda i: (0, 0)),
            pl.BlockSpec((1, _D), lambda i: (0, 0)),
            pl.BlockSpec((1, _D), lambda i: (0, 0)),
        ],
        out_specs=pl.BlockSpec((block_rows, _D), lambda i: (i, 0)),
        out_shape=jax.ShapeDtypeStruct((n, _D), jnp.float32),
        compiler_params=pltpu.CompilerParams(
            dimension_semantics=("arbitrary",),
        ),
    )(emb2d, seg3d, pos3d, st_pad, pt_pad, gamma2d, beta2d)
    return out


_NC = 2
_NS = 16
_NW = _NC * _NS
_L = 16


def _sc_row_ln(seg_v, pos_v, st_v, pt_v, g_v, b_v, emb_v, out_v, xrow_v, r):
    ridx = jnp.full((_L,), r, jnp.int32)
    sv = plsc.load_gather(seg_v, [ridx])
    pv = plsc.load_gather(pos_v, [ridx])
    col0 = lax.broadcasted_iota(jnp.int32, (_L,), 0)
    acc = None
    accq = None
    for j in range(_D // _L):
        ds = pl.ds(j * _L, _L)
        cols = col0 + (j * _L)
        x = (emb_v[r, ds] + plsc.load_gather(st_v, [sv, cols])
             + plsc.load_gather(pt_v, [pv, cols]))
        xrow_v[ds] = x
        acc = x if acc is None else acc + x
        accq = x * x if accq is None else accq + x * x
    ssum = jnp.broadcast_to(jnp.sum(acc), (_L,))
    qsum = jnp.broadcast_to(jnp.sum(accq), (_L,))
    mean = ssum * (1.0 / _D)
    var = qsum * (1.0 / _D) - mean * mean + 1e-5
    i = plsc.bitcast(var, jnp.int32)
    y = plsc.bitcast(
        jnp.full((_L,), 0x5F3759DF, jnp.int32) - lax.shift_right_logical(i, 1),
        jnp.float32)
    for _ in range(4):
        y = y * (1.5 - 0.5 * var * y * y)
    for j in range(_D // _L):
        ds = pl.ds(j * _L, _L)
        x = xrow_v[ds]
        out_v[r, ds] = (x - mean) * y * g_v[ds] + b_v[ds]


def _make_sc_kernel(n_rows, chunk_rows):
    rows_per_w = n_rows // _NW
    n_chunks = rows_per_w // chunk_rows
    mesh = plsc.VectorSubcoreMesh(core_axis_name="c", subcore_axis_name="s")

    @functools.partial(
        pl.kernel,
        out_type=jax.ShapeDtypeStruct((n_rows, _D), jnp.float32),
        mesh=mesh,
        compiler_params=pltpu.CompilerParams(needs_layout_passes=False),
        scratch_types=[
            pltpu.VMEM((6, _D), jnp.float32),
            pltpu.VMEM((20, _D), jnp.float32),
            pltpu.VMEM((_D,), jnp.float32),
            pltpu.VMEM((_D,), jnp.float32),
            pltpu.VMEM((chunk_rows, _D), jnp.float32),
            pltpu.VMEM((chunk_rows,), jnp.int32),
            pltpu.VMEM((chunk_rows,), jnp.int32),
            pltpu.VMEM((chunk_rows, _D), jnp.float32),
            pltpu.VMEM((_D,), jnp.float32),
        ],
    )
    def sc_kern(emb_hbm, seg_hbm, pos_hbm, st_hbm, pt_hbm, g_hbm, b_hbm,
                out_hbm, st_v, pt_v, g_v, b_v, emb_v, seg_v, pos_v, out_v,
                xrow_v):
        wid = lax.axis_index("s") * _NC + lax.axis_index("c")
        base = wid * rows_per_w
        pltpu.sync_copy(st_hbm, st_v)
        pltpu.sync_copy(pt_hbm, pt_v)
        pltpu.sync_copy(g_hbm, g_v)
        pltpu.sync_copy(b_hbm, b_v)

        def chunk_body(ci, carry):
            r0 = base + ci * chunk_rows
            pltpu.sync_copy(emb_hbm.at[pl.ds(r0, chunk_rows)], emb_v)
            pltpu.sync_copy(seg_hbm.at[pl.ds(r0, chunk_rows)], seg_v)
            pltpu.sync_copy(pos_hbm.at[pl.ds(r0, chunk_rows)], pos_v)

            def row_body(r, inner):
                _sc_row_ln(seg_v, pos_v, st_v, pt_v, g_v, b_v, emb_v, out_v,
                           xrow_v, r)
                return inner

            lax.fori_loop(0, chunk_rows, row_body, 0)
            pltpu.sync_copy(out_v, out_hbm.at[pl.ds(r0, chunk_rows)])
            return carry

        lax.fori_loop(0, n_chunks, chunk_body, 0)

    return sc_kern


@jax.jit
def _run_sc(emb2d, seg1d, pos1d, seg_table, pos_table, gamma1d, beta1d):
    n = emb2d.shape[0]
    sc_kern = _make_sc_kernel(n, 160)
    return sc_kern(emb2d, seg1d, pos1d, seg_table, pos_table, gamma1d, beta1d)


def kernel(embeddings, segment_types, positions, seg_table, pos_table,
           ln_gamma, ln_beta):
    b, t, d = embeddings.shape
    n = b * t
    emb2d = embeddings.reshape(n, d)
    seg1d = segment_types.astype(jnp.int32).reshape(n)
    pos1d = positions.astype(jnp.int32).reshape(n)
    out = _run_sc(emb2d, seg1d, pos1d, seg_table, pos_table,
                  ln_gamma, ln_beta)
    return out.reshape(b, t, d)


def _kernel_tc(embeddings, segment_types, positions, seg_table, pos_table,
               ln_gamma, ln_beta):
    b, t, d = embeddings.shape
    n = b * t
    block_rows = 4096
    nb = n // block_rows

    emb2d = embeddings.reshape(n, d)
    seg3d = segment_types.astype(jnp.int32).reshape(nb, block_rows, 1)
    pos3d = positions.astype(jnp.int32).reshape(nb, block_rows, 1)
    st_pad = jnp.concatenate(
        [seg_table, jnp.zeros((_SEG_PAD - seg_table.shape[0], d), jnp.float32)])
    pt_pad = jnp.concatenate(
        [pos_table, jnp.zeros((_POS_PAD - pos_table.shape[0], d), jnp.float32)])
    gamma2d = ln_gamma.reshape(1, d)
    beta2d = ln_beta.reshape(1, d)

    out = _run(emb2d, seg3d, pos3d, st_pad, pt_pad, gamma2d, beta2d,
               block_rows=block_rows)
    return out.reshape(b, t, d)


# speedup vs baseline: 1.9049x; 1.8397x over previous
import functools

import jax
import jax.numpy as jnp
from jax import lax
from jax.experimental import pallas as pl
from jax.experimental.pallas import tpu as pltpu
from jax.experimental.pallas import tpu_sc as plsc

_D = 256
_SEG_PAD = 8
_POS_PAD = 32


def _fused_kernel(emb_ref, seg_ref, pos_ref, st_ref, pt_ref, g_ref, b_ref,
                  out_ref):
    emb = emb_ref[...]
    seg = seg_ref[0]
    pos = pos_ref[0]

    r = emb.shape[0]
    oh_s = (seg == lax.broadcasted_iota(jnp.int32, (r, _SEG_PAD), 1))
    oh_p = (pos == lax.broadcasted_iota(jnp.int32, (r, _POS_PAD), 1))
    oh_s = oh_s.astype(jnp.bfloat16)
    oh_p = oh_p.astype(jnp.bfloat16)

    st = st_ref[...]
    pt = pt_ref[...]
    st_hi = st.astype(jnp.bfloat16)
    st_lo = (st - st_hi.astype(jnp.float32)).astype(jnp.bfloat16)
    pt_hi = pt.astype(jnp.bfloat16)
    pt_lo = (pt - pt_hi.astype(jnp.float32)).astype(jnp.bfloat16)

    x = emb
    x = x + jnp.dot(oh_s, st_hi, preferred_element_type=jnp.float32)
    x = x + jnp.dot(oh_s, st_lo, preferred_element_type=jnp.float32)
    x = x + jnp.dot(oh_p, pt_hi, preferred_element_type=jnp.float32)
    x = x + jnp.dot(oh_p, pt_lo, preferred_element_type=jnp.float32)

    mean = jnp.sum(x, axis=1, keepdims=True) * (1.0 / _D)
    meansq = jnp.sum(x * x, axis=1, keepdims=True) * (1.0 / _D)
    var = meansq - mean * mean
    inv = lax.rsqrt(var + 1e-5)
    gamma = g_ref[...]
    beta = b_ref[...]
    scale = inv * gamma
    shift = beta - (mean * inv) * gamma
    out_ref[...] = x * scale + shift


@functools.partial(jax.jit, static_argnames=("block_rows",))
def _run(emb2d, seg3d, pos3d, st_pad, pt_pad, gamma2d, beta2d,
         block_rows=512):
    n = emb2d.shape[0]
    nb = n // block_rows
    grid = (nb,)
    out = pl.pallas_call(
        _fused_kernel,
        grid=grid,
        in_specs=[
            pl.BlockSpec((block_rows, _D), lambda i: (i, 0)),
            pl.BlockSpec((1, block_rows, 1), lambda i: (i, 0, 0)),
            pl.BlockSpec((1, block_rows, 1), lambda i: (i, 0, 0)),
            pl.BlockSpec((_SEG_PAD, _D), lambda i: (0, 0)),
            pl.BlockSpec((_POS_PAD, _D), lambda i: (0, 0)),
            pl.BlockSpec((1, _D), lambda i: (0, 0)),
            pl.BlockSpec((1, _D), lambda i: (0, 0)),
        ],
        out_specs=pl.BlockSpec((block_rows, _D), lambda i: (i, 0)),
        out_shape=jax.ShapeDtypeStruct((n, _D), jnp.float32),
        compiler_params=pltpu.CompilerParams(
            dimension_semantics=("arbitrary",),
        ),
    )(emb2d, seg3d, pos3d, st_pad, pt_pad, gamma2d, beta2d)
    return out


_NC = 2
_NS = 16
_NW = _NC * _NS
_L = 16


def _sc_row_ln(seg_v, pos_v, st_v, pt_v, g_v, b_v, emb_v, out_v, xrow_v, r):
    ridx = jnp.full((_L,), r, jnp.int32)
    sv = plsc.load_gather(seg_v, [ridx])
    pv = plsc.load_gather(pos_v, [ridx])
    col0 = lax.broadcasted_iota(jnp.int32, (_L,), 0)
    acc = None
    accq = None
    for j in range(_D // _L):
        ds = pl.ds(j * _L, _L)
        cols = col0 + (j * _L)
        x = (emb_v[r, ds] + plsc.load_gather(st_v, [sv, cols])
             + plsc.load_gather(pt_v, [pv, cols]))
        xrow_v[ds] = x
        acc = x if acc is None else acc + x
        accq = x * x if accq is None else accq + x * x
    ssum = jnp.broadcast_to(jnp.sum(acc), (_L,))
    qsum = jnp.broadcast_to(jnp.sum(accq), (_L,))
    mean = ssum * (1.0 / _D)
    var = qsum * (1.0 / _D) - mean * mean + 1e-5
    i = plsc.bitcast(var, jnp.int32)
    y = plsc.bitcast(
        jnp.full((_L,), 0x5F3759DF, jnp.int32) - lax.shift_right_logical(i, 1),
        jnp.float32)
    for _ in range(4):
        y = y * (1.5 - 0.5 * var * y * y)
    for j in range(_D // _L):
        ds = pl.ds(j * _L, _L)
        x = xrow_v[ds]
        out_v[r, ds] = (x - mean) * y * g_v[ds] + b_v[ds]


def _make_sc_kernel(row0, n_rows, chunk_rows):
    rows_per_w = n_rows // _NW
    n_chunks = rows_per_w // chunk_rows
    mesh = plsc.VectorSubcoreMesh(core_axis_name="c", subcore_axis_name="s")

    @functools.partial(
        pl.kernel,
        out_type=jax.ShapeDtypeStruct((n_rows, _D), jnp.float32),
        mesh=mesh,
        compiler_params=pltpu.CompilerParams(needs_layout_passes=False),
        scratch_types=[
            pltpu.VMEM((6, _D), jnp.float32),
            pltpu.VMEM((20, _D), jnp.float32),
            pltpu.VMEM((_D,), jnp.float32),
            pltpu.VMEM((_D,), jnp.float32),
            pltpu.VMEM((chunk_rows, _D), jnp.float32),
            pltpu.VMEM((chunk_rows,), jnp.int32),
            pltpu.VMEM((chunk_rows,), jnp.int32),
            pltpu.VMEM((chunk_rows, _D), jnp.float32),
            pltpu.VMEM((_D,), jnp.float32),
        ],
    )
    def sc_kern(emb_hbm, seg_hbm, pos_hbm, st_hbm, pt_hbm, g_hbm, b_hbm,
                out_hbm, st_v, pt_v, g_v, b_v, emb_v, seg_v, pos_v, out_v,
                xrow_v):
        wid = lax.axis_index("s") * _NC + lax.axis_index("c")
        base = wid * rows_per_w
        pltpu.sync_copy(st_hbm, st_v)
        pltpu.sync_copy(pt_hbm, pt_v)
        pltpu.sync_copy(g_hbm, g_v)
        pltpu.sync_copy(b_hbm, b_v)

        def chunk_body(ci, carry):
            r0 = base + ci * chunk_rows
            pltpu.sync_copy(emb_hbm.at[pl.ds(row0 + r0, chunk_rows)], emb_v)
            pltpu.sync_copy(seg_hbm.at[pl.ds(row0 + r0, chunk_rows)], seg_v)
            pltpu.sync_copy(pos_hbm.at[pl.ds(row0 + r0, chunk_rows)], pos_v)

            def row_body(r, inner):
                _sc_row_ln(seg_v, pos_v, st_v, pt_v, g_v, b_v, emb_v, out_v,
                           xrow_v, r)
                return inner

            lax.fori_loop(0, chunk_rows, row_body, 0)
            pltpu.sync_copy(out_v, out_hbm.at[pl.ds(r0, chunk_rows)])
            return carry

        lax.fori_loop(0, n_chunks, chunk_body, 0)

    return sc_kern


_SC_ROWS = 57344
_SC_CHUNK = 128


@jax.jit
def _run_hybrid(emb2d, seg1d, pos1d, st_pad, pt_pad, seg_table, pos_table,
                gamma1d, beta1d):
    n = emb2d.shape[0]
    nt = n - _SC_ROWS
    block_rows = 2048
    nb = nt // block_rows

    seg3d = seg1d[:nt].reshape(nb, block_rows, 1)
    pos3d = pos1d[:nt].reshape(nb, block_rows, 1)
    gamma2d = gamma1d.reshape(1, _D)
    beta2d = beta1d.reshape(1, _D)

    tc_out = pl.pallas_call(
        _fused_kernel,
        grid=(nb,),
        in_specs=[
            pl.BlockSpec((block_rows, _D), lambda i: (i, 0)),
            pl.BlockSpec((1, block_rows, 1), lambda i: (i, 0, 0)),
            pl.BlockSpec((1, block_rows, 1), lambda i: (i, 0, 0)),
            pl.BlockSpec((_SEG_PAD, _D), lambda i: (0, 0)),
            pl.BlockSpec((_POS_PAD, _D), lambda i: (0, 0)),
            pl.BlockSpec((1, _D), lambda i: (0, 0)),
            pl.BlockSpec((1, _D), lambda i: (0, 0)),
        ],
        out_specs=pl.BlockSpec((block_rows, _D), lambda i: (i, 0)),
        out_shape=jax.ShapeDtypeStruct((nt, _D), jnp.float32),
        compiler_params=pltpu.CompilerParams(
            dimension_semantics=("arbitrary",),
        ),
    )(emb2d, seg3d, pos3d, st_pad, pt_pad, gamma2d, beta2d)

    sc_kern = _make_sc_kernel(nt, _SC_ROWS, _SC_CHUNK)
    sc_out = sc_kern(emb2d, seg1d, pos1d, seg_table, pos_table,
                     gamma1d, beta1d)
    return jnp.concatenate([tc_out, sc_out], axis=0)


def kernel(embeddings, segment_types, positions, seg_table, pos_table,
           ln_gamma, ln_beta):
    b, t, d = embeddings.shape
    n = b * t
    emb2d = embeddings.reshape(n, d)
    seg1d = segment_types.astype(jnp.int32).reshape(n)
    pos1d = positions.astype(jnp.int32).reshape(n)
    st_pad = jnp.concatenate(
        [seg_table, jnp.zeros((_SEG_PAD - seg_table.shape[0], d), jnp.float32)])
    pt_pad = jnp.concatenate(
        [pos_table, jnp.zeros((_POS_PAD - pos_table.shape[0], d), jnp.float32)])
    out = _run_hybrid(emb2d, seg1d, pos1d, st_pad, pt_pad, seg_table,
                      pos_table, ln_gamma, ln_beta)
    return out.reshape(b, t, d)


def _kernel_tc(embeddings, segment_types, positions, seg_table, pos_table,
               ln_gamma, ln_beta):
    b, t, d = embeddings.shape
    n = b * t
    block_rows = 4096
    nb = n // block_rows

    emb2d = embeddings.reshape(n, d)
    seg3d = segment_types.astype(jnp.int32).reshape(nb, block_rows, 1)
    pos3d = positions.astype(jnp.int32).reshape(nb, block_rows, 1)
    st_pad = jnp.concatenate(
        [seg_table, jnp.zeros((_SEG_PAD - seg_table.shape[0], d), jnp.float32)])
    pt_pad = jnp.concatenate(
        [pos_table, jnp.zeros((_POS_PAD - pos_table.shape[0], d), jnp.float32)])
    gamma2d = ln_gamma.reshape(1, d)
    beta2d = ln_beta.reshape(1, d)

    out = _run(emb2d, seg3d, pos3d, st_pad, pt_pad, gamma2d, beta2d,
               block_rows=block_rows)
    return out.reshape(b, t, d)


# speedup vs baseline: 1.9833x; 1.0412x over previous
import functools

import jax
import jax.numpy as jnp
from jax import lax
from jax.experimental import pallas as pl
from jax.experimental.pallas import tpu as pltpu
from jax.experimental.pallas import tpu_sc as plsc

_D = 256
_SEG_PAD = 8
_POS_PAD = 32


def _fused_kernel(emb_ref, seg_ref, pos_ref, st_ref, pt_ref, g_ref, b_ref,
                  out_ref):
    emb = emb_ref[...]
    seg = seg_ref[0]
    pos = pos_ref[0]

    r = emb.shape[0]
    oh_s = (seg == lax.broadcasted_iota(jnp.int32, (r, _SEG_PAD), 1))
    oh_p = (pos == lax.broadcasted_iota(jnp.int32, (r, _POS_PAD), 1))
    oh_s = oh_s.astype(jnp.bfloat16)
    oh_p = oh_p.astype(jnp.bfloat16)

    st = st_ref[...]
    pt = pt_ref[...]
    st_hi = st.astype(jnp.bfloat16)
    st_lo = (st - st_hi.astype(jnp.float32)).astype(jnp.bfloat16)
    pt_hi = pt.astype(jnp.bfloat16)
    pt_lo = (pt - pt_hi.astype(jnp.float32)).astype(jnp.bfloat16)

    x = emb
    x = x + jnp.dot(oh_s, st_hi, preferred_element_type=jnp.float32)
    x = x + jnp.dot(oh_s, st_lo, preferred_element_type=jnp.float32)
    x = x + jnp.dot(oh_p, pt_hi, preferred_element_type=jnp.float32)
    x = x + jnp.dot(oh_p, pt_lo, preferred_element_type=jnp.float32)

    mean = jnp.sum(x, axis=1, keepdims=True) * (1.0 / _D)
    meansq = jnp.sum(x * x, axis=1, keepdims=True) * (1.0 / _D)
    var = meansq - mean * mean
    inv = lax.rsqrt(var + 1e-5)
    gamma = g_ref[...]
    beta = b_ref[...]
    scale = inv * gamma
    shift = beta - (mean * inv) * gamma
    out_ref[...] = x * scale + shift


@functools.partial(jax.jit, static_argnames=("block_rows",))
def _run(emb2d, seg3d, pos3d, st_pad, pt_pad, gamma2d, beta2d,
         block_rows=512):
    n = emb2d.shape[0]
    nb = n // block_rows
    grid = (nb,)
    out = pl.pallas_call(
        _fused_kernel,
        grid=grid,
        in_specs=[
            pl.BlockSpec((block_rows, _D), lambda i: (i, 0)),
            pl.BlockSpec((1, block_rows, 1), lambda i: (i, 0, 0)),
            pl.BlockSpec((1, block_rows, 1), lambda i: (i, 0, 0)),
            pl.BlockSpec((_SEG_PAD, _D), lambda i: (0, 0)),
            pl.BlockSpec((_POS_PAD, _D), lambda i: (0, 0)),
            pl.BlockSpec((1, _D), lambda i: (0, 0)),
            pl.BlockSpec((1, _D), lambda i: (0, 0)),
        ],
        out_specs=pl.BlockSpec((block_rows, _D), lambda i: (i, 0)),
        out_shape=jax.ShapeDtypeStruct((n, _D), jnp.float32),
        compiler_params=pltpu.CompilerParams(
            dimension_semantics=("arbitrary",),
        ),
    )(emb2d, seg3d, pos3d, st_pad, pt_pad, gamma2d, beta2d)
    return out


_NC = 2
_NS = 16
_NW = _NC * _NS
_L = 16


_ROW_UNROLL = 4


def _sc_rows_ln(seg_v, pos_v, st_v, pt_v, g_v, b_v, emb_v, out_v, xrow_v, rg):
    col0 = lax.broadcasted_iota(jnp.int32, (_L,), 0)
    rows = []
    for u in range(_ROW_UNROLL):
        r = rg * _ROW_UNROLL + u
        ridx = jnp.full((_L,), r, jnp.int32)
        sv = plsc.load_gather(seg_v, [ridx])
        pv = plsc.load_gather(pos_v, [ridx])
        rows.append((r, sv, pv))
    stats = []
    for u, (r, sv, pv) in enumerate(rows):
        acc = None
        accq = None
        for j in range(_D // _L):
            ds = pl.ds(j * _L, _L)
            cols = col0 + (j * _L)
            x = (emb_v[r, ds] + plsc.load_gather(st_v, [sv, cols])
                 + plsc.load_gather(pt_v, [pv, cols]))
            xrow_v[u, ds] = x
            acc = x if acc is None else acc + x
            accq = x * x if accq is None else accq + x * x
        stats.append((jnp.sum(acc), jnp.sum(accq)))
    invs = []
    for u, (ssum, qsum) in enumerate(stats):
        mean = jnp.broadcast_to(ssum, (_L,)) * (1.0 / _D)
        var = (jnp.broadcast_to(qsum, (_L,)) * (1.0 / _D)
               - mean * mean + 1e-5)
        i = plsc.bitcast(var, jnp.int32)
        y = plsc.bitcast(
            jnp.full((_L,), 0x5F3759DF, jnp.int32)
            - lax.shift_right_logical(i, 1), jnp.float32)
        for _ in range(3):
            y = y * (1.5 - 0.5 * var * y * y)
        invs.append((mean, y))
    for u, ((r, _, _), (mean, y)) in enumerate(zip(rows, invs)):
        for j in range(_D // _L):
            ds = pl.ds(j * _L, _L)
            x = xrow_v[u, ds]
            out_v[r, ds] = (x - mean) * y * g_v[ds] + b_v[ds]


def _make_sc_kernel(row0, n_rows, chunk_rows):
    rows_per_w = n_rows // _NW
    n_chunks = rows_per_w // chunk_rows
    mesh = plsc.VectorSubcoreMesh(core_axis_name="c", subcore_axis_name="s")

    @functools.partial(
        pl.kernel,
        out_type=jax.ShapeDtypeStruct((n_rows, _D), jnp.float32),
        mesh=mesh,
        compiler_params=pltpu.CompilerParams(needs_layout_passes=False),
        scratch_types=[
            pltpu.VMEM((6, _D), jnp.float32),
            pltpu.VMEM((20, _D), jnp.float32),
            pltpu.VMEM((_D,), jnp.float32),
            pltpu.VMEM((_D,), jnp.float32),
            pltpu.VMEM((chunk_rows, _D), jnp.float32),
            pltpu.VMEM((chunk_rows,), jnp.int32),
            pltpu.VMEM((chunk_rows,), jnp.int32),
            pltpu.VMEM((chunk_rows, _D), jnp.float32),
            pltpu.VMEM((_ROW_UNROLL, _D), jnp.float32),
        ],
    )
    def sc_kern(emb_hbm, seg_hbm, pos_hbm, st_hbm, pt_hbm, g_hbm, b_hbm,
                out_hbm, st_v, pt_v, g_v, b_v, emb_v, seg_v, pos_v, out_v,
                xrow_v):
        wid = lax.axis_index("s") * _NC + lax.axis_index("c")
        base = wid * rows_per_w
        pltpu.sync_copy(st_hbm, st_v)
        pltpu.sync_copy(pt_hbm, pt_v)
        pltpu.sync_copy(g_hbm, g_v)
        pltpu.sync_copy(b_hbm, b_v)

        def chunk_body(ci, carry):
            r0 = base + ci * chunk_rows
            pltpu.sync_copy(emb_hbm.at[pl.ds(row0 + r0, chunk_rows)], emb_v)
            pltpu.sync_copy(seg_hbm.at[pl.ds(row0 + r0, chunk_rows)], seg_v)
            pltpu.sync_copy(pos_hbm.at[pl.ds(row0 + r0, chunk_rows)], pos_v)

            def row_body(rg, inner):
                _sc_rows_ln(seg_v, pos_v, st_v, pt_v, g_v, b_v, emb_v, out_v,
                            xrow_v, rg)
                return inner

            lax.fori_loop(0, chunk_rows // _ROW_UNROLL, row_body, 0)
            pltpu.sync_copy(out_v, out_hbm.at[pl.ds(r0, chunk_rows)])
            return carry

        lax.fori_loop(0, n_chunks, chunk_body, 0)

    return sc_kern


_SC_ROWS = 57344
_SC_CHUNK = 128


@jax.jit
def _run_hybrid(emb2d, seg1d, pos1d, st_pad, pt_pad, seg_table, pos_table,
                gamma1d, beta1d):
    n = emb2d.shape[0]
    nt = n - _SC_ROWS
    block_rows = 2048
    nb = nt // block_rows

    seg3d = seg1d[:nt].reshape(nb, block_rows, 1)
    pos3d = pos1d[:nt].reshape(nb, block_rows, 1)
    gamma2d = gamma1d.reshape(1, _D)
    beta2d = beta1d.reshape(1, _D)

    tc_out = pl.pallas_call(
        _fused_kernel,
        grid=(nb,),
        in_specs=[
            pl.BlockSpec((block_rows, _D), lambda i: (i, 0)),
            pl.BlockSpec((1, block_rows, 1), lambda i: (i, 0, 0)),
            pl.BlockSpec((1, block_rows, 1), lambda i: (i, 0, 0)),
            pl.BlockSpec((_SEG_PAD, _D), lambda i: (0, 0)),
            pl.BlockSpec((_POS_PAD, _D), lambda i: (0, 0)),
            pl.BlockSpec((1, _D), lambda i: (0, 0)),
            pl.BlockSpec((1, _D), lambda i: (0, 0)),
        ],
        out_specs=pl.BlockSpec((block_rows, _D), lambda i: (i, 0)),
        out_shape=jax.ShapeDtypeStruct((nt, _D), jnp.float32),
        compiler_params=pltpu.CompilerParams(
            dimension_semantics=("arbitrary",),
        ),
    )(emb2d, seg3d, pos3d, st_pad, pt_pad, gamma2d, beta2d)

    sc_kern = _make_sc_kernel(nt, _SC_ROWS, _SC_CHUNK)
    sc_out = sc_kern(emb2d, seg1d, pos1d, seg_table, pos_table,
                     gamma1d, beta1d)
    return jnp.concatenate([tc_out, sc_out], axis=0)


def kernel(embeddings, segment_types, positions, seg_table, pos_table,
           ln_gamma, ln_beta):
    b, t, d = embeddings.shape
    n = b * t
    emb2d = embeddings.reshape(n, d)
    seg1d = segment_types.astype(jnp.int32).reshape(n)
    pos1d = positions.astype(jnp.int32).reshape(n)
    st_pad = jnp.concatenate(
        [seg_table, jnp.zeros((_SEG_PAD - seg_table.shape[0], d), jnp.float32)])
    pt_pad = jnp.concatenate(
        [pos_table, jnp.zeros((_POS_PAD - pos_table.shape[0], d), jnp.float32)])
    out = _run_hybrid(emb2d, seg1d, pos1d, st_pad, pt_pad, seg_table,
                      pos_table, ln_gamma, ln_beta)
    return out.reshape(b, t, d)


def _kernel_tc(embeddings, segment_types, positions, seg_table, pos_table,
               ln_gamma, ln_beta):
    b, t, d = embeddings.shape
    n = b * t
    block_rows = 4096
    nb = n // block_rows

    emb2d = embeddings.reshape(n, d)
    seg3d = segment_types.astype(jnp.int32).reshape(nb, block_rows, 1)
    pos3d = positions.astype(jnp.int32).reshape(nb, block_rows, 1)
    st_pad = jnp.concatenate(
        [seg_table, jnp.zeros((_SEG_PAD - seg_table.shape[0], d), jnp.float32)])
    pt_pad = jnp.concatenate(
        [pos_table, jnp.zeros((_POS_PAD - pos_table.shape[0], d), jnp.float32)])
    gamma2d = ln_gamma.reshape(1, d)
    beta2d = ln_beta.reshape(1, d)

    out = _run(emb2d, seg3d, pos3d, st_pad, pt_pad, gamma2d, beta2d,
               block_rows=block_rows)
    return out.reshape(b, t, d)
